# baseline (device time: 32539 ns/iter reference)
import jax
import jax.numpy as jnp
from jax import lax
from jax.experimental import pallas as pl
from jax.experimental.pallas import tpu as pltpu

N_DEV = 4
B_LOC = 2
SQ = 256
SKV = 256
HQ = 16
HG = 4
DH = 64
D_MODEL = 512
GROUP = HG * DH


def kernel(x, Wq, K_ext, V_ext, Wo):
    k2 = K_ext.reshape(N_DEV * B_LOC, SKV, HQ * DH)
    v2 = V_ext.reshape(N_DEV * B_LOC, SKV, HQ * DH)

    def body(x_ref, wq_ref, wo_ref, k_any, v_any, out_ref,
             x16, wq16, wo16,
             wq_l, wo_l, wq_r, wo_r, wq_o, wo_o,
             kbuf, vbuf, bias_ref,
             send_sems, recv_sems, loc_sems):
        my_pos = lax.axis_index("i")
        right = lax.rem(my_pos + 1, N_DEV)
        left = lax.rem(my_pos + N_DEV - 1, N_DEV)
        opp = lax.rem(my_pos + 2, N_DEV)
        origins = (my_pos, left, right, opp)

        loc = []
        for c in range(N_DEV):
            col = origins[c] * GROUP
            for b in range(B_LOC):
                bg = my_pos * B_LOC + b
                for src, dst in ((k_any, kbuf), (v_any, vbuf)):
                    cp = pltpu.make_async_copy(
                        src.at[bg, :, pl.ds(col, GROUP)],
                        dst.at[c, b],
                        loc_sems.at[len(loc)])
                    cp.start()
                    loc.append(cp)

        x16[...] = x_ref[...].astype(jnp.bfloat16)
        wq16[...] = (wq_ref[...] * 0.125).astype(jnp.bfloat16)
        wo16[...] = wo_ref[...].astype(jnp.bfloat16)

        barrier = pltpu.get_barrier_semaphore()
        for nbr in (left, right):
            pl.semaphore_signal(barrier, inc=1, device_id=(nbr,),
                                device_id_type=pl.DeviceIdType.MESH)
        pl.semaphore_wait(barrier, 2)

        def rdma(src, dst, i, dev):
            return pltpu.make_async_remote_copy(
                src_ref=src, dst_ref=dst,
                send_sem=send_sems.at[i], recv_sem=recv_sems.at[i],
                device_id=(dev,), device_id_type=pl.DeviceIdType.MESH)

        t0 = rdma(wq16, wq_l, 0, right)
        t1 = rdma(wo16, wo_l, 1, right)
        t2 = rdma(wq16, wq_r, 2, left)
        t3 = rdma(wo16, wo_r, 3, left)
        for t in (t0, t1, t2, t3):
            t.start()

        qi = lax.broadcasted_iota(jnp.int32, (SQ, SKV), 0)
        ki = lax.broadcasted_iota(jnp.int32, (SQ, SKV), 1)
        mask = (jnp.abs(qi - ki) <= 128) | (ki < 32) | (qi < 32)
        bias_ref[...] = jnp.where(mask, 0.0, -1e9)

        def compute(wq_s, wo_s, c, first):
            for cp in loc[c * 2 * B_LOC:(c + 1) * 2 * B_LOC]:
                cp.wait()
            for b in range(B_LOC):
                qb = jnp.dot(x16[b], wq_s[...],
                             preferred_element_type=jnp.float32
                             ).astype(jnp.bfloat16)
                kg = kbuf[c, b].astype(jnp.bfloat16)
                vg = vbuf[c, b].astype(jnp.bfloat16)
                ctx = []
                for hh in range(HG):
                    sl = slice(hh * DH, (hh + 1) * DH)
                    q = qb[:, sl]
                    s = lax.dot_general(
                        q, kg[:, sl], (((1,), (1,)), ((), ())),
                        preferred_element_type=jnp.float32)
                    w = jnp.exp(s + bias_ref[...])
                    recip = 1.0 / jnp.sum(w, axis=-1, keepdims=True)
                    c_h = jnp.dot(w.astype(jnp.bfloat16), vg[:, sl],
                                  preferred_element_type=jnp.float32)
                    ctx.append(c_h * recip)
                ctx = jnp.concatenate(ctx, axis=1)
                contrib = jnp.dot(ctx.astype(jnp.bfloat16), wo_s[...],
                                  preferred_element_type=jnp.float32)
                if first:
                    out_ref[b] = contrib
                else:
                    out_ref[b] = out_ref[b] + contrib

        compute(wq16, wo16, 0, first=True)

        t0.wait_recv()
        t4 = rdma(wq_l, wq_o, 4, right)
        t4.start()
        t1.wait_recv()
        compute(wq_l, wo_l, 1, first=False)

        t3.wait_recv()
        t5 = rdma(wo_r, wo_o, 5, left)
        t5.start()
        t2.wait_recv()
        compute(wq_r, wo_r, 2, first=False)

        t4.wait_recv()
        t5.wait_recv()
        compute(wq_o, wo_o, 3, first=False)

        for t in (t0, t1, t2, t3, t4, t5):
            t.wait_send()

    return pl.pallas_call(
        body,
        out_shape=jax.ShapeDtypeStruct((B_LOC, SQ, D_MODEL), jnp.float32),
        in_specs=[
            pl.BlockSpec(memory_space=pltpu.VMEM),
            pl.BlockSpec(memory_space=pltpu.VMEM),
            pl.BlockSpec(memory_space=pltpu.VMEM),
            pl.BlockSpec(memory_space=pl.ANY),
            pl.BlockSpec(memory_space=pl.ANY),
        ],
        out_specs=pl.BlockSpec(memory_space=pltpu.VMEM),
        scratch_shapes=[
            pltpu.VMEM((B_LOC, SQ, D_MODEL), jnp.bfloat16),
            pltpu.VMEM((D_MODEL, GROUP), jnp.bfloat16),
            pltpu.VMEM((GROUP, D_MODEL), jnp.bfloat16),
            pltpu.VMEM((D_MODEL, GROUP), jnp.bfloat16),
            pltpu.VMEM((GROUP, D_MODEL), jnp.bfloat16),
            pltpu.VMEM((D_MODEL, GROUP), jnp.bfloat16),
            pltpu.VMEM((GROUP, D_MODEL), jnp.bfloat16),
            pltpu.VMEM((D_MODEL, GROUP), jnp.bfloat16),
            pltpu.VMEM((GROUP, D_MODEL), jnp.bfloat16),
            pltpu.VMEM((N_DEV, B_LOC, SKV, GROUP), jnp.float32),
            pltpu.VMEM((N_DEV, B_LOC, SKV, GROUP), jnp.float32),
            pltpu.VMEM((SQ, SKV), jnp.float32),
            pltpu.SemaphoreType.DMA((6,)),
            pltpu.SemaphoreType.DMA((6,)),
            pltpu.SemaphoreType.DMA((N_DEV * B_LOC * 2,)),
        ],
        compiler_params=pltpu.CompilerParams(collective_id=0),
    )(x, Wq, Wo, k2, v2)


# device time: 25350 ns/iter; 1.2836x vs baseline; 1.2836x over previous
import jax
import jax.numpy as jnp
from jax import lax
from jax.experimental import pallas as pl
from jax.experimental.pallas import tpu as pltpu

N_DEV = 4
B_LOC = 2
SQ = 256
SKV = 256
HQ = 16
HG = 4
DH = 64
D_MODEL = 512
GROUP = HG * DH


def kernel(x, Wq, K_ext, V_ext, Wo):
    my = lax.axis_index("i")
    k_loc = lax.dynamic_slice_in_dim(K_ext, my * B_LOC, B_LOC, axis=0)
    v_loc = lax.dynamic_slice_in_dim(V_ext, my * B_LOC, B_LOC, axis=0)
    k_t = jnp.transpose(k_loc.reshape(B_LOC, SKV, N_DEV, GROUP), (2, 0, 1, 3))
    v_t = jnp.transpose(v_loc.reshape(B_LOC, SKV, N_DEV, GROUP), (2, 0, 1, 3))

    def body(x_ref, wq_ref, wo_ref, k_ref, v_ref, out_ref,
             x16, wq16, wo16,
             wq_l, wo_l, wq_r, wo_r, wq_o, wo_o, bias_ref,
             send_sems, recv_sems):
        my_pos = lax.axis_index("i")
        right = lax.rem(my_pos + 1, N_DEV)
        left = lax.rem(my_pos + N_DEV - 1, N_DEV)
        opp = lax.rem(my_pos + 2, N_DEV)

        x16[...] = x_ref[...].astype(jnp.bfloat16)
        wq16[...] = (wq_ref[...] * 0.125).astype(jnp.bfloat16)
        wo16[...] = wo_ref[...].astype(jnp.bfloat16)

        barrier = pltpu.get_barrier_semaphore()
        for nbr in (left, right):
            pl.semaphore_signal(barrier, inc=1, device_id=(nbr,),
                                device_id_type=pl.DeviceIdType.MESH)
        pl.semaphore_wait(barrier, 2)

        def rdma(src, dst, i, dev):
            return pltpu.make_async_remote_copy(
                src_ref=src, dst_ref=dst,
                send_sem=send_sems.at[i], recv_sem=recv_sems.at[i],
                device_id=(dev,), device_id_type=pl.DeviceIdType.MESH)

        t0 = rdma(wq16, wq_l, 0, right)
        t1 = rdma(wo16, wo_l, 1, right)
        t2 = rdma(wq16, wq_r, 2, left)
        t3 = rdma(wo16, wo_r, 3, left)
        for t in (t0, t1, t2, t3):
            t.start()

        qi = lax.broadcasted_iota(jnp.int32, (SQ, SKV), 0)
        ki = lax.broadcasted_iota(jnp.int32, (SQ, SKV), 1)
        mask = (jnp.abs(qi - ki) <= 128) | (ki < 32) | (qi < 32)
        bias_ref[...] = jnp.where(mask, 0.0, -1e9)

        def compute(wq_s, wo_s, origin, first):
            for b in range(B_LOC):
                qb = jnp.dot(x16[b], wq_s[...],
                             preferred_element_type=jnp.float32
                             ).astype(jnp.bfloat16)
                kg = k_ref[origin, b].astype(jnp.bfloat16)
                vg = v_ref[origin, b].astype(jnp.bfloat16)
                ctx = []
                for hh in range(HG):
                    sl = slice(hh * DH, (hh + 1) * DH)
                    q = qb[:, sl]
                    s = lax.dot_general(
                        q, kg[:, sl], (((1,), (1,)), ((), ())),
                        preferred_element_type=jnp.float32)
                    w = jnp.exp(s + bias_ref[...])
                    recip = 1.0 / jnp.sum(w, axis=-1, keepdims=True)
                    c_h = jnp.dot(w.astype(jnp.bfloat16), vg[:, sl],
                                  preferred_element_type=jnp.float32)
                    ctx.append(c_h * recip)
                ctx = jnp.concatenate(ctx, axis=1)
                contrib = jnp.dot(ctx.astype(jnp.bfloat16), wo_s[...],
                                  preferred_element_type=jnp.float32)
                if first:
                    out_ref[b] = contrib
                else:
                    out_ref[b] = out_ref[b] + contrib

        compute(wq16, wo16, my_pos, first=True)

        t0.wait_recv()
        t4 = rdma(wq_l, wq_o, 4, right)
        t4.start()
        t1.wait_recv()
        compute(wq_l, wo_l, left, first=False)

        t3.wait_recv()
        t5 = rdma(wo_r, wo_o, 5, left)
        t5.start()
        t2.wait_recv()
        compute(wq_r, wo_r, right, first=False)

        t4.wait_recv()
        t5.wait_recv()
        compute(wq_o, wo_o, opp, first=False)

        for t in (t0, t1, t2, t3, t4, t5):
            t.wait_send()

    return pl.pallas_call(
        body,
        out_shape=jax.ShapeDtypeStruct((B_LOC, SQ, D_MODEL), jnp.float32),
        in_specs=[pl.BlockSpec(memory_space=pltpu.VMEM)] * 5,
        out_specs=pl.BlockSpec(memory_space=pltpu.VMEM),
        scratch_shapes=[
            pltpu.VMEM((B_LOC, SQ, D_MODEL), jnp.bfloat16),
            pltpu.VMEM((D_MODEL, GROUP), jnp.bfloat16),
            pltpu.VMEM((GROUP, D_MODEL), jnp.bfloat16),
            pltpu.VMEM((D_MODEL, GROUP), jnp.bfloat16),
            pltpu.VMEM((GROUP, D_MODEL), jnp.bfloat16),
            pltpu.VMEM((D_MODEL, GROUP), jnp.bfloat16),
            pltpu.VMEM((GROUP, D_MODEL), jnp.bfloat16),
            pltpu.VMEM((D_MODEL, GROUP), jnp.bfloat16),
            pltpu.VMEM((GROUP, D_MODEL), jnp.bfloat16),
            pltpu.VMEM((SQ, SKV), jnp.float32),
            pltpu.SemaphoreType.DMA((6,)),
            pltpu.SemaphoreType.DMA((6,)),
        ],
        compiler_params=pltpu.CompilerParams(collective_id=0),
    )(x, Wq, Wo, k_t, v_t)


# device time: 23010 ns/iter; 1.4141x vs baseline; 1.1017x over previous
import jax
import jax.numpy as jnp
from jax import lax
from jax.experimental import pallas as pl
from jax.experimental.pallas import tpu as pltpu

N_DEV = 4
B_LOC = 2
SQ = 256
SKV = 256
HQ = 16
HG = 4
DH = 64
D_MODEL = 512
GROUP = HG * DH


def kernel(x, Wq, K_ext, V_ext, Wo):
    my = lax.axis_index("i")
    k_loc = lax.dynamic_slice_in_dim(K_ext, my * B_LOC, B_LOC, axis=0)
    v_loc = lax.dynamic_slice_in_dim(V_ext, my * B_LOC, B_LOC, axis=0)
    k_t = jnp.transpose(k_loc.reshape(B_LOC, SKV, N_DEV, GROUP), (2, 0, 1, 3))
    v_t = jnp.transpose(v_loc.reshape(B_LOC, SKV, N_DEV, GROUP), (2, 0, 1, 3))

    def body(x_ref, wq_ref, wo_ref, k_ref, v_ref, out_ref,
             x16, wq16, wo16,
             wq_l, wo_l, wq_r, wo_r, wq_o, wo_o, bias_ref,
             send_sems, recv_sems):
        my_pos = lax.axis_index("i")
        right = lax.rem(my_pos + 1, N_DEV)
        left = lax.rem(my_pos + N_DEV - 1, N_DEV)
        opp = lax.rem(my_pos + 2, N_DEV)

        x16[...] = x_ref[...].astype(jnp.bfloat16)
        wq16[...] = (wq_ref[...] * 0.125).astype(jnp.bfloat16)
        wo16[...] = wo_ref[...].astype(jnp.bfloat16)

        barrier = pltpu.get_barrier_semaphore()
        for nbr in (left, right):
            pl.semaphore_signal(barrier, inc=1, device_id=(nbr,),
                                device_id_type=pl.DeviceIdType.MESH)
        pl.semaphore_wait(barrier, 2)

        def rdma(src, dst, i, dev):
            return pltpu.make_async_remote_copy(
                src_ref=src, dst_ref=dst,
                send_sem=send_sems.at[i], recv_sem=recv_sems.at[i],
                device_id=(dev,), device_id_type=pl.DeviceIdType.MESH)

        t0 = rdma(wq16, wq_l, 0, right)
        t1 = rdma(wo16, wo_l, 1, right)
        t2 = rdma(wq16, wq_r, 2, left)
        t3 = rdma(wo16, wo_r, 3, left)
        for t in (t0, t1, t2, t3):
            t.start()

        qi = lax.broadcasted_iota(jnp.int32, (SQ, SKV), 0)
        ki = lax.broadcasted_iota(jnp.int32, (SQ, SKV), 1)
        mask = (jnp.abs(qi - ki) <= 128) | (ki < 32) | (qi < 32)
        bias_ref[...] = jnp.where(mask, 0.0, -1e9)

        def attn_part(wq_s, origin):
            ctxs = []
            for b in range(B_LOC):
                qb = jnp.dot(x16[b], wq_s[...],
                             preferred_element_type=jnp.float32
                             ).astype(jnp.bfloat16)
                kg = k_ref[origin, b].astype(jnp.bfloat16)
                vg = v_ref[origin, b].astype(jnp.bfloat16)
                ctx = []
                for hh in range(HG):
                    sl = slice(hh * DH, (hh + 1) * DH)
                    q = qb[:, sl]
                    s = lax.dot_general(
                        q, kg[:, sl], (((1,), (1,)), ((), ())),
                        preferred_element_type=jnp.float32)
                    w = jnp.exp(s + bias_ref[...])
                    recip = 1.0 / jnp.sum(w, axis=-1, keepdims=True)
                    c_h = jnp.dot(w.astype(jnp.bfloat16), vg[:, sl],
                                  preferred_element_type=jnp.float32)
                    ctx.append(c_h * recip)
                ctxs.append(jnp.concatenate(ctx, axis=1)
                            .astype(jnp.bfloat16))
            return ctxs

        def out_part(wo_s, ctxs, first):
            for b in range(B_LOC):
                contrib = jnp.dot(ctxs[b], wo_s[...],
                                  preferred_element_type=jnp.float32)
                if first:
                    out_ref[b] = contrib
                else:
                    out_ref[b] = out_ref[b] + contrib

        out_part(wo16, attn_part(wq16, my_pos), first=True)

        t0.wait_recv()
        t4 = rdma(wq_l, wq_o, 4, right)
        t4.start()
        ctxs = attn_part(wq_l, left)
        t1.wait_recv()
        out_part(wo_l, ctxs, first=False)

        t2.wait_recv()
        ctxs = attn_part(wq_r, right)
        t3.wait_recv()
        t5 = rdma(wo_r, wo_o, 5, left)
        t5.start()
        out_part(wo_r, ctxs, first=False)

        t4.wait_recv()
        ctxs = attn_part(wq_o, opp)
        t5.wait_recv()
        out_part(wo_o, ctxs, first=False)

        for t in (t0, t1, t2, t3, t4, t5):
            t.wait_send()

    return pl.pallas_call(
        body,
        out_shape=jax.ShapeDtypeStruct((B_LOC, SQ, D_MODEL), jnp.float32),
        in_specs=[pl.BlockSpec(memory_space=pltpu.VMEM)] * 5,
        out_specs=pl.BlockSpec(memory_space=pltpu.VMEM),
        scratch_shapes=[
            pltpu.VMEM((B_LOC, SQ, D_MODEL), jnp.bfloat16),
            pltpu.VMEM((D_MODEL, GROUP), jnp.bfloat16),
            pltpu.VMEM((GROUP, D_MODEL), jnp.bfloat16),
            pltpu.VMEM((D_MODEL, GROUP), jnp.bfloat16),
            pltpu.VMEM((GROUP, D_MODEL), jnp.bfloat16),
            pltpu.VMEM((D_MODEL, GROUP), jnp.bfloat16),
            pltpu.VMEM((GROUP, D_MODEL), jnp.bfloat16),
            pltpu.VMEM((D_MODEL, GROUP), jnp.bfloat16),
            pltpu.VMEM((GROUP, D_MODEL), jnp.bfloat16),
            pltpu.VMEM((SQ, SKV), jnp.float32),
            pltpu.SemaphoreType.DMA((6,)),
            pltpu.SemaphoreType.DMA((6,)),
        ],
        compiler_params=pltpu.CompilerParams(collective_id=0),
    )(x, Wq, Wo, k_t, v_t)


# device time: 22945 ns/iter; 1.4181x vs baseline; 1.0028x over previous
import jax
import jax.numpy as jnp
from jax import lax
from jax.experimental import pallas as pl
from jax.experimental.pallas import tpu as pltpu

N_DEV = 4
B_LOC = 2
SQ = 256
SKV = 256
HQ = 16
HG = 4
DH = 64
D_MODEL = 512
GROUP = HG * DH


def kernel(x, Wq, K_ext, V_ext, Wo):
    my = lax.axis_index("i")
    k_loc = lax.dynamic_slice_in_dim(K_ext, my * B_LOC, B_LOC, axis=0)
    v_loc = lax.dynamic_slice_in_dim(V_ext, my * B_LOC, B_LOC, axis=0)
    k_t = k_loc.reshape(B_LOC, SKV, HQ * DH)
    v_t = v_loc.reshape(B_LOC, SKV, HQ * DH)

    def body(x_ref, wq_ref, wo_ref, k_ref, v_ref, out_ref,
             x16, wq16, wo16,
             wq_l, wo_l, wq_r, wo_r, wq_o, wo_o, bias_ref,
             send_sems, recv_sems):
        my_pos = lax.axis_index("i")
        right = lax.rem(my_pos + 1, N_DEV)
        left = lax.rem(my_pos + N_DEV - 1, N_DEV)
        opp = lax.rem(my_pos + 2, N_DEV)

        x16[...] = x_ref[...].astype(jnp.bfloat16)
        wq16[...] = (wq_ref[...] * 0.125).astype(jnp.bfloat16)
        wo16[...] = wo_ref[...].astype(jnp.bfloat16)

        barrier = pltpu.get_barrier_semaphore()
        for nbr in (left, right):
            pl.semaphore_signal(barrier, inc=1, device_id=(nbr,),
                                device_id_type=pl.DeviceIdType.MESH)
        pl.semaphore_wait(barrier, 2)

        def rdma(src, dst, i, dev):
            return pltpu.make_async_remote_copy(
                src_ref=src, dst_ref=dst,
                send_sem=send_sems.at[i], recv_sem=recv_sems.at[i],
                device_id=(dev,), device_id_type=pl.DeviceIdType.MESH)

        t0 = rdma(wq16, wq_l, 0, right)
        t1 = rdma(wo16, wo_l, 1, right)
        t2 = rdma(wq16, wq_r, 2, left)
        t3 = rdma(wo16, wo_r, 3, left)
        for t in (t0, t1, t2, t3):
            t.start()

        qi = lax.broadcasted_iota(jnp.int32, (SQ, SKV), 0)
        ki = lax.broadcasted_iota(jnp.int32, (SQ, SKV), 1)
        mask = (jnp.abs(qi - ki) <= 128) | (ki < 32) | (qi < 32)
        bias_ref[...] = jnp.where(mask, 0.0, -1e9)

        def attn_part(wq_s, origin):
            ctxs = []
            col = origin * GROUP
            for b in range(B_LOC):
                qb = jnp.dot(x16[b], wq_s[...],
                             preferred_element_type=jnp.float32
                             ).astype(jnp.bfloat16)
                kg = k_ref[b, :, pl.ds(col, GROUP)].astype(jnp.bfloat16)
                vg = v_ref[b, :, pl.ds(col, GROUP)].astype(jnp.bfloat16)
                ctx = []
                for hh in range(HG):
                    sl = slice(hh * DH, (hh + 1) * DH)
                    q = qb[:, sl]
                    s = lax.dot_general(
                        q, kg[:, sl], (((1,), (1,)), ((), ())),
                        preferred_element_type=jnp.float32)
                    w = jnp.exp(s + bias_ref[...])
                    recip = 1.0 / jnp.sum(w, axis=-1, keepdims=True)
                    c_h = jnp.dot(w.astype(jnp.bfloat16), vg[:, sl],
                                  preferred_element_type=jnp.float32)
                    ctx.append(c_h * recip)
                ctxs.append(jnp.concatenate(ctx, axis=1)
                            .astype(jnp.bfloat16))
            return ctxs

        def out_part(wo_s, ctxs, first):
            for b in range(B_LOC):
                contrib = jnp.dot(ctxs[b], wo_s[...],
                                  preferred_element_type=jnp.float32)
                if first:
                    out_ref[b] = contrib
                else:
                    out_ref[b] = out_ref[b] + contrib

        out_part(wo16, attn_part(wq16, my_pos), first=True)

        t0.wait_recv()
        t4 = rdma(wq_l, wq_o, 4, right)
        t4.start()
        ctxs = attn_part(wq_l, left)
        t1.wait_recv()
        out_part(wo_l, ctxs, first=False)

        t2.wait_recv()
        ctxs = attn_part(wq_r, right)
        t3.wait_recv()
        t5 = rdma(wo_r, wo_o, 5, left)
        t5.start()
        out_part(wo_r, ctxs, first=False)

        t4.wait_recv()
        ctxs = attn_part(wq_o, opp)
        t5.wait_recv()
        out_part(wo_o, ctxs, first=False)

        for t in (t0, t1, t2, t3, t4, t5):
            t.wait_send()

    return pl.pallas_call(
        body,
        out_shape=jax.ShapeDtypeStruct((B_LOC, SQ, D_MODEL), jnp.float32),
        in_specs=[pl.BlockSpec(memory_space=pltpu.VMEM)] * 5,
        out_specs=pl.BlockSpec(memory_space=pltpu.VMEM),
        scratch_shapes=[
            pltpu.VMEM((B_LOC, SQ, D_MODEL), jnp.bfloat16),
            pltpu.VMEM((D_MODEL, GROUP), jnp.bfloat16),
            pltpu.VMEM((GROUP, D_MODEL), jnp.bfloat16),
            pltpu.VMEM((D_MODEL, GROUP), jnp.bfloat16),
            pltpu.VMEM((GROUP, D_MODEL), jnp.bfloat16),
            pltpu.VMEM((D_MODEL, GROUP), jnp.bfloat16),
            pltpu.VMEM((GROUP, D_MODEL), jnp.bfloat16),
            pltpu.VMEM((D_MODEL, GROUP), jnp.bfloat16),
            pltpu.VMEM((GROUP, D_MODEL), jnp.bfloat16),
            pltpu.VMEM((SQ, SKV), jnp.float32),
            pltpu.SemaphoreType.DMA((6,)),
            pltpu.SemaphoreType.DMA((6,)),
        ],
        compiler_params=pltpu.CompilerParams(collective_id=0),
    )(x, Wq, Wo, k_t, v_t)


# device time: 22930 ns/iter; 1.4191x vs baseline; 1.0007x over previous
import jax
import jax.numpy as jnp
from jax import lax
from jax.experimental import pallas as pl
from jax.experimental.pallas import tpu as pltpu

N_DEV = 4
B_LOC = 2
SQ = 256
SKV = 256
HQ = 16
HG = 4
DH = 64
D_MODEL = 512
GROUP = HG * DH


def kernel(x, Wq, K_ext, V_ext, Wo):
    my = lax.axis_index("i")
    k_loc = lax.dynamic_slice_in_dim(K_ext, my * B_LOC, B_LOC, axis=0)
    v_loc = lax.dynamic_slice_in_dim(V_ext, my * B_LOC, B_LOC, axis=0)
    k_t = k_loc.reshape(B_LOC, SKV, HQ * DH)
    v_t = v_loc.reshape(B_LOC, SKV, HQ * DH)

    def body(x_ref, wq_ref, wo_ref, k_ref, v_ref, out_ref,
             x16, wq16, wo16,
             wq_l, wo_l, wq_r, wo_r, wq_o, wo_o, bias_ref,
             send_sems, recv_sems):
        my_pos = lax.axis_index("i")
        right = lax.rem(my_pos + 1, N_DEV)
        left = lax.rem(my_pos + N_DEV - 1, N_DEV)
        opp = lax.rem(my_pos + 2, N_DEV)

        x16[...] = x_ref[...].astype(jnp.bfloat16)
        wq16[...] = (wq_ref[...] * 0.125).astype(jnp.bfloat16)
        wo16[...] = wo_ref[...].astype(jnp.bfloat16)

        barrier = pltpu.get_barrier_semaphore()
        for nbr in (left, right):
            pl.semaphore_signal(barrier, inc=1, device_id=(nbr,),
                                device_id_type=pl.DeviceIdType.MESH)
        pl.semaphore_wait(barrier, 2)

        def rdma(src, dst, i, dev):
            return pltpu.make_async_remote_copy(
                src_ref=src, dst_ref=dst,
                send_sem=send_sems.at[i], recv_sem=recv_sems.at[i],
                device_id=(dev,), device_id_type=pl.DeviceIdType.MESH)

        t0 = rdma(wq16, wq_l, 0, right)
        t1 = rdma(wo16, wo_l, 1, right)
        t2 = rdma(wq16, wq_r, 2, left)
        t3 = rdma(wo16, wo_r, 3, left)
        for t in (t0, t1, t2, t3):
            t.start()

        qi = lax.broadcasted_iota(jnp.int32, (SQ, SKV), 0)
        ki = lax.broadcasted_iota(jnp.int32, (SQ, SKV), 1)
        mask = (jnp.abs(qi - ki) <= 128) | (ki < 32) | (qi < 32)
        bias_ref[...] = jnp.where(mask, 0.0, -1e9).astype(jnp.bfloat16)

        def attn_part(wq_s, origin):
            ctxs = []
            col = origin * GROUP
            for b in range(B_LOC):
                qb = jnp.dot(x16[b], wq_s[...],
                             preferred_element_type=jnp.float32
                             ).astype(jnp.bfloat16)
                kg = k_ref[b, :, pl.ds(col, GROUP)].astype(jnp.bfloat16)
                vg = v_ref[b, :, pl.ds(col, GROUP)].astype(jnp.bfloat16)
                ctx = []
                for hh in range(HG):
                    sl = slice(hh * DH, (hh + 1) * DH)
                    q = qb[:, sl]
                    s = lax.dot_general(
                        q, kg[:, sl], (((1,), (1,)), ((), ())),
                        preferred_element_type=jnp.float32)
                    w = jnp.exp(s.astype(jnp.bfloat16) + bias_ref[...])
                    recip = 1.0 / jnp.sum(w, axis=-1, keepdims=True,
                                          dtype=jnp.float32)
                    c_h = jnp.dot(w, vg[:, sl],
                                  preferred_element_type=jnp.float32)
                    ctx.append(c_h * recip)
                ctxs.append(jnp.concatenate(ctx, axis=1)
                            .astype(jnp.bfloat16))
            return ctxs

        def out_part(wo_s, ctxs, first):
            for b in range(B_LOC):
                contrib = jnp.dot(ctxs[b], wo_s[...],
                                  preferred_element_type=jnp.float32)
                if first:
                    out_ref[b] = contrib
                else:
                    out_ref[b] = out_ref[b] + contrib

        out_part(wo16, attn_part(wq16, my_pos), first=True)

        t0.wait_recv()
        t4 = rdma(wq_l, wq_o, 4, right)
        t4.start()
        ctxs = attn_part(wq_l, left)
        t1.wait_recv()
        out_part(wo_l, ctxs, first=False)

        t2.wait_recv()
        ctxs = attn_part(wq_r, right)
        t3.wait_recv()
        t5 = rdma(wo_r, wo_o, 5, left)
        t5.start()
        out_part(wo_r, ctxs, first=False)

        t4.wait_recv()
        ctxs = attn_part(wq_o, opp)
        t5.wait_recv()
        out_part(wo_o, ctxs, first=False)

        for t in (t0, t1, t2, t3, t4, t5):
            t.wait_send()

    return pl.pallas_call(
        body,
        out_shape=jax.ShapeDtypeStruct((B_LOC, SQ, D_MODEL), jnp.float32),
        in_specs=[pl.BlockSpec(memory_space=pltpu.VMEM)] * 5,
        out_specs=pl.BlockSpec(memory_space=pltpu.VMEM),
        scratch_shapes=[
            pltpu.VMEM((B_LOC, SQ, D_MODEL), jnp.bfloat16),
            pltpu.VMEM((D_MODEL, GROUP), jnp.bfloat16),
            pltpu.VMEM((GROUP, D_MODEL), jnp.bfloat16),
            pltpu.VMEM((D_MODEL, GROUP), jnp.bfloat16),
            pltpu.VMEM((GROUP, D_MODEL), jnp.bfloat16),
            pltpu.VMEM((D_MODEL, GROUP), jnp.bfloat16),
            pltpu.VMEM((GROUP, D_MODEL), jnp.bfloat16),
            pltpu.VMEM((D_MODEL, GROUP), jnp.bfloat16),
            pltpu.VMEM((GROUP, D_MODEL), jnp.bfloat16),
            pltpu.VMEM((SQ, SKV), jnp.bfloat16),
            pltpu.SemaphoreType.DMA((6,)),
            pltpu.SemaphoreType.DMA((6,)),
        ],
        compiler_params=pltpu.CompilerParams(collective_id=0),
    )(x, Wq, Wo, k_t, v_t)


# device time: 22142 ns/iter; 1.4696x vs baseline; 1.0356x over previous
import jax
import jax.numpy as jnp
from jax import lax
from jax.experimental import pallas as pl
from jax.experimental.pallas import tpu as pltpu

N_DEV = 4
B_LOC = 2
SQ = 256
SKV = 256
HQ = 16
HG = 4
DH = 64
D_MODEL = 512
GROUP = HG * DH


def kernel(x, Wq, K_ext, V_ext, Wo):
    my = lax.axis_index("i")
    k_loc = lax.dynamic_slice_in_dim(K_ext, my * B_LOC, B_LOC, axis=0)
    v_loc = lax.dynamic_slice_in_dim(V_ext, my * B_LOC, B_LOC, axis=0)
    k_t = k_loc.reshape(B_LOC, SKV, HQ * DH)
    v_t = v_loc.reshape(B_LOC, SKV, HQ * DH)

    def body(x_ref, wq_ref, wo_ref, k_ref, v_ref, out_ref,
             x16, wq16, wo16,
             wq_l, wo_l, wq_r, wo_r, wq_o, wo_o, bias_ref,
             send_sems, recv_sems):
        my_pos = lax.axis_index("i")
        right = lax.rem(my_pos + 1, N_DEV)
        left = lax.rem(my_pos + N_DEV - 1, N_DEV)
        opp = lax.rem(my_pos + 2, N_DEV)

        barrier = pltpu.get_barrier_semaphore()
        for nbr in (left, right):
            pl.semaphore_signal(barrier, inc=1, device_id=(nbr,),
                                device_id_type=pl.DeviceIdType.MESH)

        x16[...] = x_ref[...].astype(jnp.bfloat16)
        wq16[...] = (wq_ref[...] * 0.125).astype(jnp.bfloat16)
        wo16[...] = wo_ref[...].astype(jnp.bfloat16)

        pl.semaphore_wait(barrier, 2)

        def rdma(src, dst, i, dev):
            return pltpu.make_async_remote_copy(
                src_ref=src, dst_ref=dst,
                send_sem=send_sems.at[i], recv_sem=recv_sems.at[i],
                device_id=(dev,), device_id_type=pl.DeviceIdType.MESH)

        t0 = rdma(wq16, wq_l, 0, right)
        t1 = rdma(wo16, wo_l, 1, right)
        t2 = rdma(wq16, wq_r, 2, left)
        t3 = rdma(wo16, wo_r, 3, left)
        for t in (t0, t1, t2, t3):
            t.start()

        qi = lax.broadcasted_iota(jnp.int32, (SQ, SKV), 0)
        ki = lax.broadcasted_iota(jnp.int32, (SQ, SKV), 1)
        mask = (jnp.abs(qi - ki) <= 128) | (ki < 32) | (qi < 32)
        bias_ref[...] = jnp.where(mask, 0.0, -1e9).astype(jnp.bfloat16)

        def attn_one(wq_s, origin, b):
            col = origin * GROUP
            qb = jnp.dot(x16[b], wq_s[...],
                         preferred_element_type=jnp.float32
                         ).astype(jnp.bfloat16)
            kg = k_ref[b, :, pl.ds(col, GROUP)].astype(jnp.bfloat16)
            vg = v_ref[b, :, pl.ds(col, GROUP)].astype(jnp.bfloat16)
            ctx = []
            for hh in range(HG):
                sl = slice(hh * DH, (hh + 1) * DH)
                q = qb[:, sl]
                s = lax.dot_general(
                    q, kg[:, sl], (((1,), (1,)), ((), ())),
                    preferred_element_type=jnp.float32)
                w = jnp.exp(s.astype(jnp.bfloat16) + bias_ref[...])
                recip = 1.0 / jnp.sum(w, axis=-1, keepdims=True,
                                      dtype=jnp.float32)
                c_h = jnp.dot(w, vg[:, sl],
                              preferred_element_type=jnp.float32)
                ctx.append(c_h * recip)
            return jnp.concatenate(ctx, axis=1).astype(jnp.bfloat16)

        def attn_part(wq_s, origin):
            return [attn_one(wq_s, origin, b) for b in range(B_LOC)]

        def out_part(wo_s, ctxs, first):
            for b in range(B_LOC):
                contrib = jnp.dot(ctxs[b], wo_s[...],
                                  preferred_element_type=jnp.float32)
                if first:
                    out_ref[b] = contrib
                else:
                    out_ref[b] = out_ref[b] + contrib

        out_part(wo16, attn_part(wq16, my_pos), first=True)

        t0.wait_recv()
        t4 = rdma(wq_l, wq_o, 4, right)
        t4.start()
        ctxs = attn_part(wq_l, left)
        t1.wait_recv()
        out_part(wo_l, ctxs, first=False)

        t2.wait_recv()
        ctx0 = attn_one(wq_r, right, 0)
        t3.wait_recv()
        t5 = rdma(wo_r, wo_o, 5, left)
        t5.start()
        ctx1 = attn_one(wq_r, right, 1)
        out_part(wo_r, [ctx0, ctx1], first=False)

        t4.wait_recv()
        ctxs = attn_part(wq_o, opp)
        t5.wait_recv()
        out_part(wo_o, ctxs, first=False)

        for t in (t0, t1, t2, t3, t4, t5):
            t.wait_send()

    return pl.pallas_call(
        body,
        out_shape=jax.ShapeDtypeStruct((B_LOC, SQ, D_MODEL), jnp.float32),
        in_specs=[pl.BlockSpec(memory_space=pltpu.VMEM)] * 5,
        out_specs=pl.BlockSpec(memory_space=pltpu.VMEM),
        scratch_shapes=[
            pltpu.VMEM((B_LOC, SQ, D_MODEL), jnp.bfloat16),
            pltpu.VMEM((D_MODEL, GROUP), jnp.bfloat16),
            pltpu.VMEM((GROUP, D_MODEL), jnp.bfloat16),
            pltpu.VMEM((D_MODEL, GROUP), jnp.bfloat16),
            pltpu.VMEM((GROUP, D_MODEL), jnp.bfloat16),
            pltpu.VMEM((D_MODEL, GROUP), jnp.bfloat16),
            pltpu.VMEM((GROUP, D_MODEL), jnp.bfloat16),
            pltpu.VMEM((D_MODEL, GROUP), jnp.bfloat16),
            pltpu.VMEM((GROUP, D_MODEL), jnp.bfloat16),
            pltpu.VMEM((SQ, SKV), jnp.bfloat16),
            pltpu.SemaphoreType.DMA((6,)),
            pltpu.SemaphoreType.DMA((6,)),
        ],
        compiler_params=pltpu.CompilerParams(collective_id=0),
    )(x, Wq, Wo, k_t, v_t)


# device time: 20830 ns/iter; 1.5621x vs baseline; 1.0630x over previous
import jax
import jax.numpy as jnp
from jax import lax
from jax.experimental import pallas as pl
from jax.experimental.pallas import tpu as pltpu

N_DEV = 4
B_LOC = 2
SQ = 256
SKV = 256
HQ = 16
HG = 4
DH = 64
D_MODEL = 512
GROUP = HG * DH


def kernel(x, Wq, K_ext, V_ext, Wo):
    my = lax.axis_index("i")
    k_loc = lax.dynamic_slice_in_dim(K_ext, my * B_LOC, B_LOC, axis=0)
    v_loc = lax.dynamic_slice_in_dim(V_ext, my * B_LOC, B_LOC, axis=0)
    k_t = k_loc.reshape(B_LOC, SKV, HQ * DH).astype(jnp.bfloat16)
    v_t = v_loc.reshape(B_LOC, SKV, HQ * DH).astype(jnp.bfloat16)

    def body(x_ref, wq_ref, wo_ref, k_ref, v_ref, out_ref,
             x16, wq16, wo16,
             wq_l, wo_l, wq_r, wo_r, wq_o, wo_o, bias_ref,
             send_sems, recv_sems):
        my_pos = lax.axis_index("i")
        right = lax.rem(my_pos + 1, N_DEV)
        left = lax.rem(my_pos + N_DEV - 1, N_DEV)
        opp = lax.rem(my_pos + 2, N_DEV)

        barrier = pltpu.get_barrier_semaphore()
        for nbr in (left, right):
            pl.semaphore_signal(barrier, inc=1, device_id=(nbr,),
                                device_id_type=pl.DeviceIdType.MESH)

        x16[...] = x_ref[...].astype(jnp.bfloat16)
        wq16[...] = (wq_ref[...] * 0.125).astype(jnp.bfloat16)
        wo16[...] = wo_ref[...].astype(jnp.bfloat16)

        pl.semaphore_wait(barrier, 2)

        def rdma(src, dst, i, dev):
            return pltpu.make_async_remote_copy(
                src_ref=src, dst_ref=dst,
                send_sem=send_sems.at[i], recv_sem=recv_sems.at[i],
                device_id=(dev,), device_id_type=pl.DeviceIdType.MESH)

        t0 = rdma(wq16, wq_l, 0, right)
        t1 = rdma(wo16, wo_l, 1, right)
        t2 = rdma(wq16, wq_r, 2, left)
        t3 = rdma(wo16, wo_r, 3, left)
        for t in (t0, t1, t2, t3):
            t.start()

        qi = lax.broadcasted_iota(jnp.int32, (SQ, SKV), 0)
        ki = lax.broadcasted_iota(jnp.int32, (SQ, SKV), 1)
        mask = (jnp.abs(qi - ki) <= 128) | (ki < 32) | (qi < 32)
        bias_ref[...] = jnp.where(mask, 0.0, -1e9).astype(jnp.bfloat16)

        def attn_one(wq_s, origin, b):
            col = origin * GROUP
            qb = jnp.dot(x16[b], wq_s[...],
                         preferred_element_type=jnp.float32
                         ).astype(jnp.bfloat16)
            kg = k_ref[b, :, pl.ds(col, GROUP)]
            vg = v_ref[b, :, pl.ds(col, GROUP)]
            ctx = []
            for hh in range(HG):
                sl = slice(hh * DH, (hh + 1) * DH)
                q = qb[:, sl]
                s = lax.dot_general(
                    q, kg[:, sl], (((1,), (1,)), ((), ())),
                    preferred_element_type=jnp.float32)
                w = jnp.exp(s.astype(jnp.bfloat16) + bias_ref[...])
                recip = 1.0 / jnp.sum(w, axis=-1, keepdims=True,
                                      dtype=jnp.float32)
                c_h = jnp.dot(w, vg[:, sl],
                              preferred_element_type=jnp.float32)
                ctx.append(c_h * recip)
            return jnp.concatenate(ctx, axis=1).astype(jnp.bfloat16)

        def attn_part(wq_s, origin):
            return [attn_one(wq_s, origin, b) for b in range(B_LOC)]

        def out_part(wo_s, ctxs, first):
            for b in range(B_LOC):
                contrib = jnp.dot(ctxs[b], wo_s[...],
                                  preferred_element_type=jnp.float32)
                if first:
                    out_ref[b] = contrib
                else:
                    out_ref[b] = out_ref[b] + contrib

        out_part(wo16, attn_part(wq16, my_pos), first=True)

        t0.wait_recv()
        t4 = rdma(wq_l, wq_o, 4, right)
        t4.start()
        ctxs = attn_part(wq_l, left)
        t1.wait_recv()
        out_part(wo_l, ctxs, first=False)

        t2.wait_recv()
        ctx0 = attn_one(wq_r, right, 0)
        t3.wait_recv()
        t5 = rdma(wo_r, wo_o, 5, left)
        t5.start()
        ctx1 = attn_one(wq_r, right, 1)
        out_part(wo_r, [ctx0, ctx1], first=False)

        t4.wait_recv()
        ctxs = attn_part(wq_o, opp)
        t5.wait_recv()
        out_part(wo_o, ctxs, first=False)

        for t in (t0, t1, t2, t3, t4, t5):
            t.wait_send()

    return pl.pallas_call(
        body,
        out_shape=jax.ShapeDtypeStruct((B_LOC, SQ, D_MODEL), jnp.float32),
        in_specs=[pl.BlockSpec(memory_space=pltpu.VMEM)] * 5,
        out_specs=pl.BlockSpec(memory_space=pltpu.VMEM),
        scratch_shapes=[
            pltpu.VMEM((B_LOC, SQ, D_MODEL), jnp.bfloat16),
            pltpu.VMEM((D_MODEL, GROUP), jnp.bfloat16),
            pltpu.VMEM((GROUP, D_MODEL), jnp.bfloat16),
            pltpu.VMEM((D_MODEL, GROUP), jnp.bfloat16),
            pltpu.VMEM((GROUP, D_MODEL), jnp.bfloat16),
            pltpu.VMEM((D_MODEL, GROUP), jnp.bfloat16),
            pltpu.VMEM((GROUP, D_MODEL), jnp.bfloat16),
            pltpu.VMEM((D_MODEL, GROUP), jnp.bfloat16),
            pltpu.VMEM((GROUP, D_MODEL), jnp.bfloat16),
            pltpu.VMEM((SQ, SKV), jnp.bfloat16),
            pltpu.SemaphoreType.DMA((6,)),
            pltpu.SemaphoreType.DMA((6,)),
        ],
        compiler_params=pltpu.CompilerParams(collective_id=0),
    )(x, Wq, Wo, k_t, v_t)
